# Initial kernel scaffold; baseline (speedup 1.0000x reference)
import jax
import jax.numpy as jnp
from jax.experimental import pallas as pl


def kernel(img_index, table):
    raise NotImplementedError("write your pallas kernel here")


# speedup vs baseline: 3.4590x; 3.4590x over previous
import functools

import jax
import jax.numpy as jnp
from jax import lax
from jax.experimental import pallas as pl
from jax.experimental.pallas import tpu as pltpu
from jax.experimental.pallas import tpu_sc as plsc

_NUM_TABLES = 1000
_NUM_LAYER = 12
_BATCH = 4096
_D = _NUM_LAYER * 32 * 32
_NC, _NS = 2, 16
_NW = _NC * _NS
_B_PER_W = _BATCH // _NW
_K = 4
_N_CHUNK = _B_PER_W // _K


def _make_gather():
    mesh = plsc.VectorSubcoreMesh(core_axis_name="c", subcore_axis_name="s")

    @functools.partial(
        pl.kernel,
        mesh=mesh,
        out_type=jax.ShapeDtypeStruct((_BATCH, _D), jnp.float32),
        scratch_types=[
            pltpu.VMEM((_N_CHUNK, _K), jnp.int32),
            pltpu.VMEM((_K, _D), jnp.float32),
            pltpu.SemaphoreType.DMA,
        ],
    )
    def gather_kernel(idx_hbm, table_hbm, out_hbm, idx_v, buf, gsem):
        wid = lax.axis_index("s") * _NC + lax.axis_index("c")
        pltpu.sync_copy(idx_hbm.at[wid], idx_v)
        base = wid * _B_PER_W

        def body(j, carry):
            pltpu.async_copy(table_hbm.at[idx_v.at[j]], buf, gsem).wait()
            pltpu.sync_copy(buf, out_hbm.at[pl.ds(base + j * _K, _K)])
            return carry

        lax.fori_loop(0, _N_CHUNK, body, 0)

    return gather_kernel


_gather = _make_gather()


def kernel(img_index, table):
    idx3 = img_index.reshape(_NW, _N_CHUNK, _K)
    table2 = table.reshape(_NUM_TABLES, _D)
    out = _gather(idx3, table2)
    return out.reshape(_BATCH, _NUM_LAYER, 32, 32)


# speedup vs baseline: 3.6167x; 1.0456x over previous
import functools

import jax
import jax.numpy as jnp
from jax import lax
from jax.experimental import pallas as pl
from jax.experimental.pallas import tpu as pltpu
from jax.experimental.pallas import tpu_sc as plsc

_NUM_TABLES = 1000
_NUM_LAYER = 12
_BATCH = 4096
_D = _NUM_LAYER * 32 * 32
_NC, _NS = 2, 16
_NW = _NC * _NS
_B_PER_W = _BATCH // _NW
_K = 4
_N_CHUNK = _B_PER_W // _K


def _make_gather():
    mesh = plsc.VectorSubcoreMesh(core_axis_name="c", subcore_axis_name="s")

    @functools.partial(
        pl.kernel,
        mesh=mesh,
        out_type=jax.ShapeDtypeStruct((_BATCH, _D), jnp.float32),
        scratch_types=[
            pltpu.VMEM((_N_CHUNK, _K), jnp.int32),
            pltpu.VMEM((_K, _D), jnp.float32),
            pltpu.VMEM((_K, _D), jnp.float32),
            pltpu.SemaphoreType.DMA,
            pltpu.SemaphoreType.DMA,
            pltpu.SemaphoreType.DMA,
            pltpu.SemaphoreType.DMA,
        ],
    )
    def gather_kernel(idx_hbm, table_hbm, out_hbm, idx_v,
                      buf0, buf1, gsem0, gsem1, osem0, osem1):
        wid = lax.axis_index("s") * _NC + lax.axis_index("c")
        pltpu.sync_copy(idx_hbm.at[wid], idx_v)
        base = wid * _B_PER_W
        bufs = (buf0, buf1)
        gsems = (gsem0, gsem1)
        osems = (osem0, osem1)

        def wait_gather(p):
            pltpu.make_async_copy(
                table_hbm.at[idx_v.at[0]], bufs[p], gsems[p]).wait()

        def wait_out(p):
            pltpu.make_async_copy(
                bufs[p], out_hbm.at[pl.ds(0, _K)], osems[p]).wait()

        def start_gather(j, p):
            pltpu.async_copy(table_hbm.at[idx_v.at[j]], bufs[p], gsems[p])

        def start_out(j, p):
            pltpu.async_copy(bufs[p], out_hbm.at[pl.ds(base + j * _K, _K)],
                             osems[p])

        start_gather(0, 0)
        wait_gather(0)
        start_out(0, 0)
        start_gather(1, 1)

        def body(i, carry):
            j0 = 2 * i + 1
            wait_gather(1)
            start_out(j0, 1)
            wait_out(0)
            start_gather(j0 + 1, 0)
            wait_gather(0)
            start_out(j0 + 1, 0)
            wait_out(1)
            start_gather(j0 + 2, 1)
            return carry

        lax.fori_loop(0, _N_CHUNK // 2 - 1, body, 0)

        j_last = _N_CHUNK - 1
        wait_gather(1)
        start_out(j_last, 1)
        wait_out(0)
        wait_out(1)

    return gather_kernel


_gather = _make_gather()


def kernel(img_index, table):
    idx3 = img_index.reshape(_NW, _N_CHUNK, _K)
    table2 = table.reshape(_NUM_TABLES, _D)
    out = _gather(idx3, table2)
    return out.reshape(_BATCH, _NUM_LAYER, 32, 32)
